# baseline (speedup 1.0000x reference)
import functools

import jax
import jax.numpy as jnp
from jax import lax
from jax.experimental import pallas as pl
from jax.experimental.pallas import tpu as pltpu
from jax.experimental.pallas import tpu_sc as plsc

D_MODEL = 1024
SCALE = 32.0
NUM_WORKERS = 32
LANES = 16
CHUNK = 16
NBUF = 3


def _emb_body(idx_hbm, table_hbm, out_hbm, idx_v, *rest):
    gbufs = rest[0:NBUF]
    sbufs = rest[NBUF : 2 * NBUF]
    gsems = rest[2 * NBUF : 3 * NBUF]
    osems = rest[3 * NBUF : 4 * NBUF]

    nbatch, seq = idx_hbm.shape
    rows_per_w = (nbatch * seq) // NUM_WORKERS
    w_per_row = seq // rows_per_w
    nchunk = rows_per_w // CHUNK
    ngroup = nchunk // NBUF
    wid = lax.axis_index("s") * 2 + lax.axis_index("c")
    brow = wid // w_per_row
    col0 = (wid % w_per_row) * rows_per_w
    pltpu.sync_copy(idx_hbm.at[brow, pl.ds(col0, rows_per_w)], idx_v)

    def gstart(b, c):
        off = pl.multiple_of(c * CHUNK, 8)
        pltpu.async_copy(table_hbm.at[idx_v.at[pl.ds(off, CHUNK)]], gbufs[b], gsems[b])

    def gwait(b):
        pltpu.make_async_copy(table_hbm.at[pl.ds(0, CHUNK)], gbufs[b], gsems[b]).wait()

    def ostart(b, c):
        off = pl.multiple_of(c * CHUNK, 8)
        pltpu.async_copy(
            sbufs[b], out_hbm.at[brow, pl.ds(col0 + off, CHUNK)], osems[b]
        )

    def owait(b):
        pltpu.make_async_copy(
            sbufs[b], out_hbm.at[0, pl.ds(0, CHUNK)], osems[b]
        ).wait()

    def scale(b):
        def row_body(r2, c):
            r = r2 * 2
            for rr in (0, 1):
                for j in range(D_MODEL // LANES):
                    sl = pl.ds(j * LANES, LANES)
                    sbufs[b][r + rr, sl] = gbufs[b][r + rr, sl] * SCALE
            return c

        lax.fori_loop(0, CHUNK // 2, row_body, 0)

    for b in range(NBUF):
        gstart(b, b)

    for b in range(NBUF):
        gwait(b)
        scale(b)
        ostart(b, b)
        gstart(b, NBUF + b)

    def group(gi, carry):
        c0 = gi * NBUF
        for b in range(NBUF):
            c = c0 + b
            gwait(b)
            owait(b)
            scale(b)
            ostart(b, c)

            @pl.when(c + NBUF < nchunk)
            def _():
                gstart(b, c + NBUF)

        return carry

    lax.fori_loop(1, ngroup, group, 0)

    for c in range(ngroup * NBUF, nchunk):
        b = c % NBUF
        gwait(b)
        owait(b)
        scale(b)
        ostart(b, c)

    for b in range(NBUF):
        owait(b)


def kernel(x, embedding_table):
    nbatch, seq = x.shape
    idx = x.astype(jnp.int32)
    mesh = plsc.VectorSubcoreMesh(core_axis_name="c", subcore_axis_name="s")
    rows_per_w = (nbatch * seq) // NUM_WORKERS
    scratch = (
        [pltpu.VMEM((rows_per_w,), jnp.int32)]
        + [pltpu.VMEM((CHUNK, D_MODEL), jnp.float32) for _ in range(2 * NBUF)]
        + [pltpu.SemaphoreType.DMA for _ in range(2 * NBUF)]
    )
    out = pl.kernel(
        _emb_body,
        out_type=jax.ShapeDtypeStruct((nbatch, seq, D_MODEL), jnp.float32),
        mesh=mesh,
        scratch_types=scratch,
    )(idx, embedding_table)
    return out


# speedup vs baseline: 1.1501x; 1.1501x over previous
import functools

import jax
import jax.numpy as jnp
from jax import lax
from jax.experimental import pallas as pl
from jax.experimental.pallas import tpu as pltpu
from jax.experimental.pallas import tpu_sc as plsc

D_MODEL = 1024
SCALE = 32.0
NUM_WORKERS = 32
LANES = 16
CHUNK = 16
NBUF = 3


def _emb_body(idx_hbm, table_hbm, out_hbm, idx_v, *rest):
    gbufs = rest[0:NBUF]
    sbufs = rest[NBUF : 2 * NBUF]
    gsems = rest[2 * NBUF : 3 * NBUF]
    osems = rest[3 * NBUF : 4 * NBUF]

    nbatch, seq = idx_hbm.shape
    rows_per_w = (nbatch * seq) // NUM_WORKERS
    w_per_row = seq // rows_per_w
    nchunk = rows_per_w // CHUNK
    ngroup = nchunk // NBUF
    wid = lax.axis_index("s") * 2 + lax.axis_index("c")
    brow = wid // w_per_row
    col0 = (wid % w_per_row) * rows_per_w
    pltpu.sync_copy(idx_hbm.at[brow, pl.ds(col0, rows_per_w)], idx_v)

    def gstart(b, c):
        off = pl.multiple_of(c * CHUNK, 8)
        pltpu.async_copy(table_hbm.at[idx_v.at[pl.ds(off, CHUNK)]], gbufs[b], gsems[b])

    def gwait(b):
        pltpu.make_async_copy(table_hbm.at[pl.ds(0, CHUNK)], gbufs[b], gsems[b]).wait()

    def ostart(b, c):
        off = pl.multiple_of(c * CHUNK, 8)
        pltpu.async_copy(
            sbufs[b], out_hbm.at[brow, pl.ds(col0 + off, CHUNK)], osems[b]
        )

    def owait(b):
        pltpu.make_async_copy(
            sbufs[b], out_hbm.at[0, pl.ds(0, CHUNK)], osems[b]
        ).wait()

    def scale(b):
        @plsc.parallel_loop(0, CHUNK)
        def _(r):
            for j in range(D_MODEL // LANES):
                sl = pl.ds(j * LANES, LANES)
                sbufs[b][r, sl] = gbufs[b][r, sl] * SCALE

    for b in range(NBUF):
        gstart(b, b)

    for b in range(NBUF):
        gwait(b)
        scale(b)
        ostart(b, b)
        gstart(b, NBUF + b)

    def group(gi, carry):
        c0 = gi * NBUF
        for b in range(NBUF):
            c = c0 + b
            gwait(b)
            owait(b)
            scale(b)
            ostart(b, c)

            @pl.when(c + NBUF < nchunk)
            def _():
                gstart(b, c + NBUF)

        return carry

    lax.fori_loop(1, ngroup, group, 0)

    for c in range(ngroup * NBUF, nchunk):
        b = c % NBUF
        gwait(b)
        owait(b)
        scale(b)
        ostart(b, c)

    for b in range(NBUF):
        owait(b)


def kernel(x, embedding_table):
    nbatch, seq = x.shape
    idx = x.astype(jnp.int32)
    mesh = plsc.VectorSubcoreMesh(core_axis_name="c", subcore_axis_name="s")
    rows_per_w = (nbatch * seq) // NUM_WORKERS
    scratch = (
        [pltpu.VMEM((rows_per_w,), jnp.int32)]
        + [pltpu.VMEM((CHUNK, D_MODEL), jnp.float32) for _ in range(2 * NBUF)]
        + [pltpu.SemaphoreType.DMA for _ in range(2 * NBUF)]
    )
    out = pl.kernel(
        _emb_body,
        out_type=jax.ShapeDtypeStruct((nbatch, seq, D_MODEL), jnp.float32),
        mesh=mesh,
        scratch_types=scratch,
    )(idx, embedding_table)
    return out


# speedup vs baseline: 1.1687x; 1.0162x over previous
import functools

import jax
import jax.numpy as jnp
from jax import lax
from jax.experimental import pallas as pl
from jax.experimental.pallas import tpu as pltpu
from jax.experimental.pallas import tpu_sc as plsc

D_MODEL = 1024
SCALE = 32.0
NUM_WORKERS = 32
LANES = 16
CHUNK = 16
NBUF = 3


def _emb_body(idx_hbm, table_hbm, out_hbm, idx_v, *rest):
    gbufs = rest[0:NBUF]
    sbufs = rest[NBUF : 2 * NBUF]
    gsems = rest[2 * NBUF : 3 * NBUF]
    osems = rest[3 * NBUF : 4 * NBUF]

    nbatch, seq = idx_hbm.shape
    rows_per_w = (nbatch * seq) // NUM_WORKERS
    w_per_row = seq // rows_per_w
    nchunk = rows_per_w // CHUNK
    ngroup = nchunk // NBUF
    wid = lax.axis_index("s") * 2 + lax.axis_index("c")
    brow = wid // w_per_row
    col0 = (wid % w_per_row) * rows_per_w
    pltpu.sync_copy(idx_hbm.at[brow, pl.ds(col0, rows_per_w)], idx_v)

    def gstart(b, c):
        off = pl.multiple_of(c * CHUNK, 8)
        pltpu.async_copy(table_hbm.at[idx_v.at[pl.ds(off, CHUNK)]], gbufs[b], gsems[b])

    def gwait(b):
        pltpu.make_async_copy(table_hbm.at[pl.ds(0, CHUNK)], gbufs[b], gsems[b]).wait()

    def ostart(b, c):
        off = pl.multiple_of(c * CHUNK, 8)
        pltpu.async_copy(
            sbufs[b], out_hbm.at[brow, pl.ds(col0 + off, CHUNK)], osems[b]
        )

    def owait(b):
        pltpu.make_async_copy(
            sbufs[b], out_hbm.at[0, pl.ds(0, CHUNK)], osems[b]
        ).wait()

    def scale(b):
        def row_body(r, c):
            for j in range(D_MODEL // LANES):
                sl = pl.ds(j * LANES, LANES)
                sbufs[b][r, sl] = gbufs[b][r, sl] * SCALE
            return c

        lax.fori_loop(0, CHUNK, row_body, 0)

    for b in range(NBUF):
        gstart(b, b)

    for b in range(NBUF):
        gwait(b)
        scale(b)
        ostart(b, b)
        gstart(b, NBUF + b)

    def group(gi, carry):
        c0 = gi * NBUF
        for b in range(NBUF):
            c = c0 + b
            gwait(b)
            owait(b)
            scale(b)
            ostart(b, c)

            @pl.when(c + NBUF < nchunk)
            def _():
                gstart(b, c + NBUF)

        return carry

    lax.fori_loop(1, ngroup, group, 0)

    for c in range(ngroup * NBUF, nchunk):
        b = c % NBUF
        gwait(b)
        owait(b)
        scale(b)
        ostart(b, c)

    for b in range(NBUF):
        owait(b)


def kernel(x, embedding_table):
    nbatch, seq = x.shape
    idx = x.astype(jnp.int32)
    mesh = plsc.VectorSubcoreMesh(core_axis_name="c", subcore_axis_name="s")
    rows_per_w = (nbatch * seq) // NUM_WORKERS
    scratch = (
        [pltpu.VMEM((rows_per_w,), jnp.int32)]
        + [pltpu.VMEM((CHUNK, D_MODEL), jnp.float32) for _ in range(2 * NBUF)]
        + [pltpu.SemaphoreType.DMA for _ in range(2 * NBUF)]
    )
    out = pl.kernel(
        _emb_body,
        out_type=jax.ShapeDtypeStruct((nbatch, seq, D_MODEL), jnp.float32),
        mesh=mesh,
        scratch_types=scratch,
    )(idx, embedding_table)
    return out
